# Initial kernel scaffold; baseline (speedup 1.0000x reference)
import jax
import jax.numpy as jnp
from jax.experimental import pallas as pl


def kernel(x_s, x_t, edge_index, edge_attr, u, batch_s, batch_t, W1, b1, W2, b2):
    raise NotImplementedError("write your pallas kernel here")


# speedup vs baseline: 3.9589x; 3.9589x over previous
import jax
import jax.numpy as jnp
from jax import lax
from jax.experimental import pallas as pl
from jax.experimental.pallas import tpu as pltpu
from jax.experimental.pallas import tpu_sc as plsc

F_S = 10
F_T = 5
F_OUT = 10
B = 1024
NC = 2
NS = 16
BLK = 128
CHUNK_BLKS = 16
SEG_ROWS = B // NS


def _zero_2d(ref, rows, cols):
  zero16 = jnp.zeros((16,), jnp.float32)
  for j in range((rows * cols) // 16):
    flat = lax.iota(jnp.int32, 16) + j * 16
    plsc.store_scatter(ref, [flat // cols, flat % cols], zero16)


def _seg_body(xs_hbm, segs_hbm, xt_hbm, segt_hbm,
              s8_o, sm_o, t8_o,
              f8_buf, fs_buf, ft_buf, idx_buf, m8, t8, zs,
              acc_s8, acc_sm, acc_t8):
  cid = lax.axis_index("c")
  sid = lax.axis_index("s")
  r0 = sid * SEG_ROWS
  iota16 = lax.iota(jnp.int32, 16)

  _zero_2d(zs, SEG_ROWS, 8)
  pltpu.sync_copy(zs, acc_s8.at[pl.ds(r0, SEG_ROWS), :])
  pltpu.sync_copy(zs, acc_sm.at[pl.ds(r0, SEG_ROWS), :])
  pltpu.sync_copy(zs, acc_t8.at[pl.ds(r0, SEG_ROWS), :])
  _zero_2d(m8, BLK, 8)
  _zero_2d(t8, BLK, 8)
  one16 = jnp.ones((16,), jnp.float32)
  for k in range(8):
    rows = iota16 + k * 16
    plsc.store_scatter(m8, [rows, jnp.full((16,), 2, jnp.int32)], one16)
    plsc.store_scatter(t8, [rows, jnp.full((16,), 5, jnp.int32)], one16)
  plsc.subcore_barrier()

  def partition(total_blocks):
    half = total_blocks // NC
    per_tile = half // NS
    extra = half - per_tile * NS
    blk0 = cid * half + sid * per_tile + jnp.minimum(sid, extra)
    return blk0, per_tile, extra

  def s_block(j, fb_off):
    pltpu.sync_copy(f8_buf.at[pl.ds(fb_off, BLK), :],
                    acc_s8.at[idx_buf.at[j, 0]], add=True)
    for c in (8, 9):
      cc = jnp.full((16,), c, jnp.int32)
      dc = jnp.full((16,), c - 8, jnp.int32)
      for k in range(8):
        rows = iota16 + k * 16
        v = plsc.load_gather(fs_buf, [rows + fb_off, cc])
        plsc.store_scatter(m8, [rows, dc], v)
    pltpu.sync_copy(m8, acc_sm.at[idx_buf.at[j, 0]], add=True)

  blk0, per_tile, extra = partition(segs_hbm.shape[0])
  n_chunks = per_tile // CHUNK_BLKS
  tail = per_tile - n_chunks * CHUNK_BLKS

  def s_chunk(cj, _):
    b0 = blk0 + cj * CHUNK_BLKS
    pltpu.sync_copy(xs_hbm.at[pl.ds(b0 * BLK, CHUNK_BLKS * BLK), pl.ds(0, 8)],
                    f8_buf)
    pltpu.sync_copy(xs_hbm.at[pl.ds(b0 * BLK, CHUNK_BLKS * BLK), :], fs_buf)
    pltpu.sync_copy(segs_hbm.at[pl.ds(b0, CHUNK_BLKS), :, :], idx_buf)

    def blk(j, _):
      s_block(j, j * BLK)
      return 0
    lax.fori_loop(0, CHUNK_BLKS, blk, 0)
    return 0
  lax.fori_loop(0, n_chunks, s_chunk, 0)

  def s_tail_block(bi):
    pltpu.sync_copy(xs_hbm.at[pl.ds(bi * BLK, BLK), pl.ds(0, 8)],
                    f8_buf.at[pl.ds(0, BLK), :])
    pltpu.sync_copy(xs_hbm.at[pl.ds(bi * BLK, BLK), :],
                    fs_buf.at[pl.ds(0, BLK), :])
    pltpu.sync_copy(segs_hbm.at[pl.ds(bi, 1), :, :], idx_buf.at[pl.ds(0, 1), :, :])
    s_block(0, 0)

  def s_tail(tj, _):
    s_tail_block(blk0 + n_chunks * CHUNK_BLKS + tj)
    return 0
  lax.fori_loop(0, tail, s_tail, 0)

  @pl.when(sid < extra)
  def _():
    s_tail_block(blk0 + per_tile)

  def t_block(j, fb_off):
    for c in range(5):
      cc = jnp.full((16,), c, jnp.int32)
      for k in range(8):
        rows = iota16 + k * 16
        v = plsc.load_gather(ft_buf, [rows + fb_off, cc])
        plsc.store_scatter(t8, [rows, cc], v)
    pltpu.sync_copy(t8, acc_t8.at[idx_buf.at[j, 0]], add=True)

  blk0, per_tile, extra = partition(segt_hbm.shape[0])
  n_chunks = per_tile // CHUNK_BLKS
  tail = per_tile - n_chunks * CHUNK_BLKS

  def t_chunk(cj, _):
    b0 = blk0 + cj * CHUNK_BLKS
    pltpu.sync_copy(xt_hbm.at[pl.ds(b0 * BLK, CHUNK_BLKS * BLK), :], ft_buf)
    pltpu.sync_copy(segt_hbm.at[pl.ds(b0, CHUNK_BLKS), :, :], idx_buf)

    def blk(j, _):
      t_block(j, j * BLK)
      return 0
    lax.fori_loop(0, CHUNK_BLKS, blk, 0)
    return 0
  lax.fori_loop(0, n_chunks, t_chunk, 0)

  def t_tail_block(bi):
    pltpu.sync_copy(xt_hbm.at[pl.ds(bi * BLK, BLK), :],
                    ft_buf.at[pl.ds(0, BLK), :])
    pltpu.sync_copy(segt_hbm.at[pl.ds(bi, 1), :, :], idx_buf.at[pl.ds(0, 1), :, :])
    t_block(0, 0)

  def t_tail(tj, _):
    t_tail_block(blk0 + n_chunks * CHUNK_BLKS + tj)
    return 0
  lax.fori_loop(0, tail, t_tail, 0)

  @pl.when(sid < extra)
  def _():
    t_tail_block(blk0 + per_tile)

  plsc.subcore_barrier()

  out_r0 = cid * B + r0
  for acc, out in ((acc_s8, s8_o), (acc_sm, sm_o), (acc_t8, t8_o)):
    pltpu.sync_copy(acc.at[pl.ds(r0, SEG_ROWS), :], zs)
    pltpu.sync_copy(zs, out.at[pl.ds(out_r0, SEG_ROWS), :])


def _segment_sums(x_s, segs2, x_t, segt2):
  mesh = plsc.VectorSubcoreMesh(core_axis_name="c", subcore_axis_name="s",
                                num_cores=NC, num_subcores=NS)
  kern = pl.kernel(
      _seg_body,
      out_type=[
          jax.ShapeDtypeStruct((NC * B, 8), jnp.float32),
          jax.ShapeDtypeStruct((NC * B, 8), jnp.float32),
          jax.ShapeDtypeStruct((NC * B, 8), jnp.float32),
      ],
      mesh=mesh,
      compiler_params=pltpu.CompilerParams(needs_layout_passes=False,
                                           use_tc_tiling_on_sc=False),
      scratch_types=[
          pltpu.VMEM((CHUNK_BLKS * BLK, 8), jnp.float32),
          pltpu.VMEM((CHUNK_BLKS * BLK, F_S), jnp.float32),
          pltpu.VMEM((CHUNK_BLKS * BLK, F_T), jnp.float32),
          pltpu.VMEM((CHUNK_BLKS, 1, BLK), jnp.int32),
          pltpu.VMEM((BLK, 8), jnp.float32),
          pltpu.VMEM((BLK, 8), jnp.float32),
          pltpu.VMEM((SEG_ROWS, 8), jnp.float32),
          pltpu.VMEM_SHARED((B, 8), jnp.float32),
          pltpu.VMEM_SHARED((B, 8), jnp.float32),
          pltpu.VMEM_SHARED((B, 8), jnp.float32),
      ],
  )
  return kern(x_s, segs2, x_t, segt2)


def _mlp_body(s8, sm, t8, u, w1, b1, w2, b2, o):
  s8c = s8[0] + s8[1]
  smc = sm[0] + sm[1]
  t8c = t8[0] + t8[1]
  s_sum = jnp.concatenate([s8c, smc[:, 0:2]], axis=1)
  s_cnt = jnp.maximum(smc[:, 2:3], 1.0)
  t_sum = t8c[:, 0:5]
  t_cnt = jnp.maximum(t8c[:, 5:6], 1.0)
  h = jnp.concatenate([u[...], s_sum / s_cnt, t_sum / t_cnt], axis=1)
  h1 = jnp.dot(h, w1[...], preferred_element_type=jnp.float32) + b1[...]
  h1 = jnp.where(h1 >= 0, h1, 0.1 * h1)
  o[...] = jnp.dot(h1, w2[...], preferred_element_type=jnp.float32) + b2[...]


def _mlp(s8, sm, t8, u, W1, b1, W2, b2):
  return pl.pallas_call(
      _mlp_body,
      out_shape=jax.ShapeDtypeStruct((B, F_OUT), jnp.float32),
  )(s8.reshape(NC, B, 8), sm.reshape(NC, B, 8), t8.reshape(NC, B, 8),
    u, W1, b1.reshape(1, -1), W2, b2.reshape(1, -1))


@jax.jit
def kernel(x_s, x_t, edge_index, edge_attr, u, batch_s, batch_t,
           W1, b1, W2, b2):
  del edge_index, edge_attr
  segs2 = batch_s.astype(jnp.int32).reshape(-1, 1, BLK)
  segt2 = batch_t.astype(jnp.int32).reshape(-1, 1, BLK)
  s8, sm, t8 = _segment_sums(x_s, segs2, x_t, segt2)
  return _mlp(s8, sm, t8, u, W1, b1, W2, b2)


# speedup vs baseline: 5.5577x; 1.4038x over previous
import jax
import jax.numpy as jnp
from jax import lax
from jax.experimental import pallas as pl
from jax.experimental.pallas import tpu as pltpu
from jax.experimental.pallas import tpu_sc as plsc

F_S = 10
F_T = 5
F_OUT = 10
B = 1024
NC = 2
NS = 16
BLK = 128
CHUNK_BLKS = 16
SEG_ROWS = B // NS


def _zero_2d(ref, rows, cols):
  zero16 = jnp.zeros((16,), jnp.float32)
  for j in range((rows * cols) // 16):
    flat = lax.iota(jnp.int32, 16) + j * 16
    plsc.store_scatter(ref, [flat // cols, flat % cols], zero16)


def _seg_body(xs_hbm, segs_hbm, xt_hbm, segt_hbm,
              s8_o, sm_o, t8_o,
              fs_buf, ft_buf, idx_buf, s8b, m8, t8, zs,
              acc_s8, acc_sm, acc_t8):
  cid = lax.axis_index("c")
  sid = lax.axis_index("s")
  r0 = sid * SEG_ROWS
  iota16 = lax.iota(jnp.int32, 16)
  rows_k = [iota16 + 16 * k for k in range(8)]
  rk10 = [r * F_S for r in rows_k]
  rk5 = [r * F_T for r in rows_k]

  _zero_2d(zs, SEG_ROWS, 8)
  pltpu.sync_copy(zs, acc_s8.at[pl.ds(r0, SEG_ROWS), :])
  pltpu.sync_copy(zs, acc_sm.at[pl.ds(r0, SEG_ROWS), :])
  pltpu.sync_copy(zs, acc_t8.at[pl.ds(r0, SEG_ROWS), :])
  _zero_2d(m8, BLK, 8)
  _zero_2d(t8, BLK, 8)
  one16 = jnp.ones((16,), jnp.float32)
  for k in range(8):
    plsc.store_scatter(m8, [rows_k[k], jnp.full((16,), 2, jnp.int32)], one16)
    plsc.store_scatter(t8, [rows_k[k], jnp.full((16,), 5, jnp.int32)], one16)
  plsc.subcore_barrier()

  def partition(total_blocks):
    half = total_blocks // NC
    per_tile = half // NS
    extra = half - per_tile * NS
    blk0 = cid * half + sid * per_tile + jnp.minimum(sid, extra)
    return blk0, per_tile, extra

  def s_block(j, off):
    for c in range(F_S):
      dst, dc = (s8b, c) if c < 8 else (m8, c - 8)
      cc = jnp.full((16,), dc, jnp.int32)
      for k in range(8):
        v = plsc.load_gather(fs_buf, [rk10[k] + (off + c)])
        plsc.store_scatter(dst, [rows_k[k], cc], v)
    pltpu.sync_copy(s8b, acc_s8.at[idx_buf.at[j, 0]], add=True)
    pltpu.sync_copy(m8, acc_sm.at[idx_buf.at[j, 0]], add=True)

  blk0, per_tile, extra = partition(segs_hbm.shape[0])
  n_chunks = per_tile // CHUNK_BLKS
  tail = per_tile - n_chunks * CHUNK_BLKS

  def s_chunk(cj, _):
    b0 = blk0 + cj * CHUNK_BLKS
    pltpu.sync_copy(xs_hbm.at[pl.ds(b0 * BLK * F_S, CHUNK_BLKS * BLK * F_S)],
                    fs_buf)
    pltpu.sync_copy(segs_hbm.at[pl.ds(b0, CHUNK_BLKS), :, :], idx_buf)

    def blk(j, _):
      s_block(j, j * BLK * F_S)
      return 0
    lax.fori_loop(0, CHUNK_BLKS, blk, 0)
    return 0
  lax.fori_loop(0, n_chunks, s_chunk, 0)

  def s_tail_block(bi):
    pltpu.sync_copy(xs_hbm.at[pl.ds(bi * BLK * F_S, BLK * F_S)],
                    fs_buf.at[pl.ds(0, BLK * F_S)])
    pltpu.sync_copy(segs_hbm.at[pl.ds(bi, 1), :, :], idx_buf.at[pl.ds(0, 1), :, :])
    s_block(0, 0)

  def s_tail(tj, _):
    s_tail_block(blk0 + n_chunks * CHUNK_BLKS + tj)
    return 0
  lax.fori_loop(0, tail, s_tail, 0)

  @pl.when(sid < extra)
  def _():
    s_tail_block(blk0 + per_tile)

  def t_block(j, off):
    for c in range(F_T):
      cc = jnp.full((16,), c, jnp.int32)
      for k in range(8):
        v = plsc.load_gather(ft_buf, [rk5[k] + (off + c)])
        plsc.store_scatter(t8, [rows_k[k], cc], v)
    pltpu.sync_copy(t8, acc_t8.at[idx_buf.at[j, 0]], add=True)

  blk0, per_tile, extra = partition(segt_hbm.shape[0])
  n_chunks = per_tile // CHUNK_BLKS
  tail = per_tile - n_chunks * CHUNK_BLKS

  def t_chunk(cj, _):
    b0 = blk0 + cj * CHUNK_BLKS
    pltpu.sync_copy(xt_hbm.at[pl.ds(b0 * BLK * F_T, CHUNK_BLKS * BLK * F_T)],
                    ft_buf)
    pltpu.sync_copy(segt_hbm.at[pl.ds(b0, CHUNK_BLKS), :, :], idx_buf)

    def blk(j, _):
      t_block(j, j * BLK * F_T)
      return 0
    lax.fori_loop(0, CHUNK_BLKS, blk, 0)
    return 0
  lax.fori_loop(0, n_chunks, t_chunk, 0)

  def t_tail_block(bi):
    pltpu.sync_copy(xt_hbm.at[pl.ds(bi * BLK * F_T, BLK * F_T)],
                    ft_buf.at[pl.ds(0, BLK * F_T)])
    pltpu.sync_copy(segt_hbm.at[pl.ds(bi, 1), :, :], idx_buf.at[pl.ds(0, 1), :, :])
    t_block(0, 0)

  def t_tail(tj, _):
    t_tail_block(blk0 + n_chunks * CHUNK_BLKS + tj)
    return 0
  lax.fori_loop(0, tail, t_tail, 0)

  @pl.when(sid < extra)
  def _():
    t_tail_block(blk0 + per_tile)

  plsc.subcore_barrier()

  out_r0 = cid * B + r0
  for acc, out in ((acc_s8, s8_o), (acc_sm, sm_o), (acc_t8, t8_o)):
    pltpu.sync_copy(acc.at[pl.ds(r0, SEG_ROWS), :], zs)
    pltpu.sync_copy(zs, out.at[pl.ds(out_r0, SEG_ROWS), :])


def _segment_sums(xs_flat, segs2, xt_flat, segt2):
  mesh = plsc.VectorSubcoreMesh(core_axis_name="c", subcore_axis_name="s",
                                num_cores=NC, num_subcores=NS)
  kern = pl.kernel(
      _seg_body,
      out_type=[
          jax.ShapeDtypeStruct((NC * B, 8), jnp.float32),
          jax.ShapeDtypeStruct((NC * B, 8), jnp.float32),
          jax.ShapeDtypeStruct((NC * B, 8), jnp.float32),
      ],
      mesh=mesh,
      compiler_params=pltpu.CompilerParams(needs_layout_passes=False,
                                           use_tc_tiling_on_sc=False),
      scratch_types=[
          pltpu.VMEM((CHUNK_BLKS * BLK * F_S,), jnp.float32),
          pltpu.VMEM((CHUNK_BLKS * BLK * F_T,), jnp.float32),
          pltpu.VMEM((CHUNK_BLKS, 1, BLK), jnp.int32),
          pltpu.VMEM((BLK, 8), jnp.float32),
          pltpu.VMEM((BLK, 8), jnp.float32),
          pltpu.VMEM((BLK, 8), jnp.float32),
          pltpu.VMEM((SEG_ROWS, 8), jnp.float32),
          pltpu.VMEM_SHARED((B, 8), jnp.float32),
          pltpu.VMEM_SHARED((B, 8), jnp.float32),
          pltpu.VMEM_SHARED((B, 8), jnp.float32),
      ],
  )
  return kern(xs_flat, segs2, xt_flat, segt2)


def _mlp_body(s8, sm, t8, u, w1, b1, w2, b2, o):
  s8c = s8[0] + s8[1]
  smc = sm[0] + sm[1]
  t8c = t8[0] + t8[1]
  s_sum = jnp.concatenate([s8c, smc[:, 0:2]], axis=1)
  s_cnt = jnp.maximum(smc[:, 2:3], 1.0)
  t_sum = t8c[:, 0:5]
  t_cnt = jnp.maximum(t8c[:, 5:6], 1.0)
  h = jnp.concatenate([u[...], s_sum / s_cnt, t_sum / t_cnt], axis=1)
  h1 = jnp.dot(h, w1[...], preferred_element_type=jnp.float32) + b1[...]
  h1 = jnp.where(h1 >= 0, h1, 0.1 * h1)
  o[...] = jnp.dot(h1, w2[...], preferred_element_type=jnp.float32) + b2[...]


def _mlp(s8, sm, t8, u, W1, b1, W2, b2):
  return pl.pallas_call(
      _mlp_body,
      out_shape=jax.ShapeDtypeStruct((B, F_OUT), jnp.float32),
  )(s8.reshape(NC, B, 8), sm.reshape(NC, B, 8), t8.reshape(NC, B, 8),
    u, W1, b1.reshape(1, -1), W2, b2.reshape(1, -1))


@jax.jit
def kernel(x_s, x_t, edge_index, edge_attr, u, batch_s, batch_t,
           W1, b1, W2, b2):
  del edge_index, edge_attr
  segs2 = batch_s.astype(jnp.int32).reshape(-1, 1, BLK)
  segt2 = batch_t.astype(jnp.int32).reshape(-1, 1, BLK)
  s8, sm, t8 = _segment_sums(x_s.reshape(-1), segs2, x_t.reshape(-1), segt2)
  return _mlp(s8, sm, t8, u, W1, b1, W2, b2)
